# Initial kernel scaffold; baseline (speedup 1.0000x reference)
import jax
import jax.numpy as jnp
from jax.experimental import pallas as pl


def kernel(input_val):
    raise NotImplementedError("write your pallas kernel here")


# speedup vs baseline: 1.8230x; 1.8230x over previous
import functools
import math

import jax
import jax.numpy as jnp
from jax.experimental import pallas as pl
from jax.experimental.pallas import tpu as pltpu

_TEMPERATURE = 0.1
_INV_T = 1.0 / _TEMPERATURE
_INV_T_LOG2E = _INV_T * math.log2(math.e)
_EPS = 1e-8


def _normalize_body(x_ref, xn_ref):
    x = x_ref[...]
    nrm = jnp.sqrt(jnp.sum(x * x, axis=1, keepdims=True))
    nrm = jnp.maximum(nrm, _EPS)
    xn_ref[...] = (x * (1.0 / nrm)).astype(jnp.bfloat16)


def _lane_fold(a, bn):
    acc = a[:, 0:128]
    for k in range(1, bn // 128):
        acc = acc + a[:, k * 128:(k + 1) * 128]
    return acc


def _ntx_body(n, bm, bn, rows_ref, cols_ref, dacc_ref, pacc_ref):
    i = pl.program_id(0)
    j = pl.program_id(1)
    s = jax.lax.dot_general(
        rows_ref[...], cols_ref[...],
        dimension_numbers=(((1,), (1,)), ((), ())),
        preferred_element_type=jnp.float32,
    )
    e = jnp.exp2(s * _INV_T_LOG2E)
    part = _lane_fold(e, bn)

    @pl.when(j == 0)
    def _init():
        dacc_ref[...] = part
        pacc_ref[...] = jnp.zeros_like(part)

    @pl.when(j != 0)
    def _acc():
        dacc_ref[...] = dacc_ref[...] + part

    @pl.when(j == (i * bm) // bn)
    def _diag():
        row_ids = i * bm + jax.lax.broadcasted_iota(jnp.int32, (bm, bn), 0)
        col_ids = j * bn + jax.lax.broadcasted_iota(jnp.int32, (bm, bn), 1)
        masked = jnp.where(col_ids == row_ids, e, 0.0)
        dacc_ref[...] = dacc_ref[...] - _lane_fold(masked, bn)

    @pl.when(j == (i * bm + n) // bn)
    def _pos():
        row_ids = i * bm + jax.lax.broadcasted_iota(jnp.int32, (bm, bn), 0)
        col_ids = j * bn + jax.lax.broadcasted_iota(jnp.int32, (bm, bn), 1)
        masked = jnp.where(col_ids == row_ids + n, s * _INV_T, 0.0)
        pacc_ref[...] = pacc_ref[...] + _lane_fold(masked, bn)


def _loss_body(batch, dacc_ref, pacc_ref, out_ref):
    d = jnp.sum(dacc_ref[...], axis=1, keepdims=True)
    p = jnp.sum(pacc_ref[...], axis=1, keepdims=True)
    lt = jnp.log(d) - p
    out_ref[...] = jnp.sum(lt, axis=0, keepdims=True) * (1.0 / batch)


def kernel(input_val):
    B, D = input_val.shape
    n = B // 2
    RB = min(256, B)
    BM = min(512, n)
    BN = min(1024, B)

    xn = pl.pallas_call(
        _normalize_body,
        grid=(B // RB,),
        in_specs=[pl.BlockSpec((RB, D), lambda i: (i, 0))],
        out_specs=pl.BlockSpec((RB, D), lambda i: (i, 0)),
        out_shape=jax.ShapeDtypeStruct((B, D), jnp.bfloat16),
        compiler_params=pltpu.CompilerParams(
            dimension_semantics=("parallel",)),
    )(input_val)

    dacc, pacc = pl.pallas_call(
        functools.partial(_ntx_body, n, BM, BN),
        grid=(n // BM, B // BN),
        in_specs=[
            pl.BlockSpec((BM, D), lambda i, j: (i, 0)),
            pl.BlockSpec((BN, D), lambda i, j: (j, 0)),
        ],
        out_specs=[
            pl.BlockSpec((BM, 128), lambda i, j: (i, 0)),
            pl.BlockSpec((BM, 128), lambda i, j: (i, 0)),
        ],
        out_shape=[
            jax.ShapeDtypeStruct((n, 128), jnp.float32),
            jax.ShapeDtypeStruct((n, 128), jnp.float32),
        ],
        compiler_params=pltpu.CompilerParams(
            dimension_semantics=("parallel", "arbitrary")),
    )(xn, xn)

    loss = pl.pallas_call(
        functools.partial(_loss_body, B),
        in_specs=[
            pl.BlockSpec((n, 128), lambda: (0, 0)),
            pl.BlockSpec((n, 128), lambda: (0, 0)),
        ],
        out_specs=pl.BlockSpec((1, 1), lambda: (0, 0)),
        out_shape=jax.ShapeDtypeStruct((1, 1), jnp.float32),
    )(dacc, pacc)
    return loss[0, 0]
